# baseline (speedup 1.0000x reference)
import functools
import math

import jax
import jax.numpy as jnp
from jax import lax
from jax.experimental import pallas as pl
from jax.experimental.pallas import tpu as pltpu
from jax.experimental.pallas import tpu_sc as plsc

_D = 128
_L = 2048
_B = 4
_NC = 2
_NS = 16
_NW = _NC * _NS
_PPW = _L // _NW
_LANES = 16
_SCALE = math.sqrt(float(_D))

_mesh = plsc.VectorSubcoreMesh(core_axis_name="c", subcore_axis_name="s")


@jax.jit
def _embed_sc(idx_flat, table, pe2):
    @functools.partial(
        pl.kernel,
        out_type=jax.ShapeDtypeStruct((_B, _L, _D), jnp.float32),
        mesh=_mesh,
        scratch_types=[
            pltpu.VMEM((_B * _PPW,), jnp.int32),
            [pltpu.VMEM((_PPW, _D), jnp.float32) for _ in range(_B)],
            [pltpu.VMEM((_PPW, _D), jnp.float32) for _ in range(_B)],
            [pltpu.SemaphoreType.DMA for _ in range(_B)],
            pltpu.SemaphoreType.DMA,
            pltpu.SemaphoreType.DMA,
        ],
    )
    def k(table_hbm, idx_hbm, pe_hbm, out_hbm,
          idx_v, gbufs, obufs, gsems, csem, osem):
        wid = lax.axis_index("s") * _NC + lax.axis_index("c")
        p0 = wid * _PPW

        pe_copies = [
            pltpu.async_copy(pe_hbm.at[pl.ds(p0, _PPW)], obufs[b], csem)
            for b in range(_B)
        ]
        pltpu.sync_copy(idx_hbm.at[pl.ds(wid * _B * _PPW, _B * _PPW)], idx_v)
        gathers = [
            pltpu.async_copy(
                table_hbm.at[idx_v.at[pl.ds(b * _PPW, _PPW)]],
                gbufs[b],
                gsems[b],
            )
            for b in range(_B)
        ]
        for c in pe_copies:
            c.wait()

        stores = []
        for b in range(_B):
            gathers[b].wait()
            gb, ob = gbufs[b], obufs[b]

            @pl.loop(0, _PPW, step=8)
            def _(r, gb=gb, ob=ob):
                for dr in range(8):
                    for c in range(0, _D, _LANES):
                        sl = (r + dr, pl.ds(c, _LANES))
                        plsc.addupdate(ob.at[sl], gb[sl] * _SCALE)

            stores.append(
                pltpu.async_copy(ob, out_hbm.at[b, pl.ds(p0, _PPW)], osem)
            )
        for st in stores:
            st.wait()

    return k(table, idx_flat, pe2)


def kernel(src, table, pe):
    idx_flat = src.reshape(_B, _NW, _PPW).transpose(1, 0, 2).reshape(-1)
    pe2 = pe.reshape(pe.shape[1], _D)[:_L]
    return _embed_sc(idx_flat, table, pe2)


# speedup vs baseline: 1.0796x; 1.0796x over previous
import functools
import math

import jax
import jax.numpy as jnp
from jax import lax
from jax.experimental import pallas as pl
from jax.experimental.pallas import tpu as pltpu
from jax.experimental.pallas import tpu_sc as plsc

_D = 128
_L = 2048
_B = 4
_NC = 2
_NS = 16
_NW = _NC * _NS
_PPW = _L // _NW
_LANES = 16
_SCALE = math.sqrt(float(_D))

_mesh = plsc.VectorSubcoreMesh(core_axis_name="c", subcore_axis_name="s")


@jax.jit
def _embed_sc(idx_flat, table, pe2):
    @functools.partial(
        pl.kernel,
        out_type=jax.ShapeDtypeStruct((_B, _L, _D), jnp.float32),
        mesh=_mesh,
        scratch_types=[
            pltpu.VMEM((_B * _PPW,), jnp.int32),
            [pltpu.VMEM((_PPW, _D), jnp.float32) for _ in range(_B)],
            [pltpu.VMEM((_PPW, _D), jnp.float32) for _ in range(_B)],
            [pltpu.SemaphoreType.DMA for _ in range(_B)],
            pltpu.SemaphoreType.DMA,
            pltpu.SemaphoreType.DMA,
        ],
    )
    def k(table_hbm, idx_hbm, pe_hbm, out_hbm,
          idx_v, gbufs, obufs, gsems, csem, osem):
        wid = lax.axis_index("s") * _NC + lax.axis_index("c")
        p0 = wid * _PPW

        pe_copies = [
            pltpu.async_copy(pe_hbm.at[pl.ds(p0, _PPW)], obufs[b], csem)
            for b in range(_B)
        ]
        pltpu.sync_copy(idx_hbm.at[pl.ds(wid * _B * _PPW, _B * _PPW)], idx_v)
        gathers = [
            pltpu.async_copy(
                table_hbm.at[idx_v.at[pl.ds(b * _PPW, _PPW)]],
                gbufs[b],
                gsems[b],
            )
            for b in range(_B)
        ]
        for c in pe_copies:
            c.wait()

        stores = []
        for b in range(_B):
            gathers[b].wait()
            gb, ob = gbufs[b], obufs[b]

            @pl.loop(0, _PPW)
            def _(r, gb=gb, ob=ob):
                for c in range(0, _D, _LANES):
                    sl = (r, pl.ds(c, _LANES))
                    plsc.addupdate(ob.at[sl], gb[sl] * _SCALE)

            stores.append(
                pltpu.async_copy(ob, out_hbm.at[b, pl.ds(p0, _PPW)], osem)
            )
        for st in stores:
            st.wait()

    return k(table, idx_flat, pe2)


def kernel(src, table, pe):
    idx_flat = src.reshape(_B, _NW, _PPW).transpose(1, 0, 2).reshape(-1)
    pe2 = pe.reshape(pe.shape[1], _D)[:_L]
    return _embed_sc(idx_flat, table, pe2)
